# Initial kernel scaffold; baseline (speedup 1.0000x reference)
import jax
import jax.numpy as jnp
from jax.experimental import pallas as pl


def kernel(x, edge_attr, edge_index, W_proj, b_proj, W_tag, b_tag, W_g1, b_g1, W_g2, b_g2, W_ih, W_hh, b_ih, b_hh, W_s, b_s, prelu_a):
    raise NotImplementedError("write your pallas kernel here")


# speedup vs baseline: 18.6251x; 18.6251x over previous
import functools

import jax
import jax.numpy as jnp
from jax import lax
from jax.experimental import pallas as pl
from jax.experimental.pallas import tpu as pltpu
from jax.experimental.pallas import tpu_sc as plsc

N = 10000
NPAD = 10240
E = 320000
H = 20
DP = 32
NC = 2
NS = 16
NW = NC * NS
EPW = E // NW
EB = 400
NBLK = EPW // EB
RPS = NPAD // NS

_sc_mesh = plsc.VectorSubcoreMesh(core_axis_name="c", subcore_axis_name="s")


@functools.partial(
    pl.kernel,
    out_type=jax.ShapeDtypeStruct((NC, 2, NPAD), jnp.float32),
    mesh=_sc_mesh,
    scratch_types=[
        pltpu.VMEM_SHARED((NPAD,), jnp.float32),
        pltpu.VMEM_SHARED((NPAD,), jnp.float32),
        pltpu.VMEM((EB,), jnp.int32),
        pltpu.VMEM((EB,), jnp.int32),
        pltpu.VMEM((EB,), jnp.float32),
    ],
)
def _degrees_sc(src_hbm, dst_hbm, zeros_hbm, out_hbm,
                acc_o, acc_i, idx_s, idx_d, ones_v):
    cid = lax.axis_index("c")
    sid = lax.axis_index("s")

    def fill(i, carry):
        ones_v[pl.ds(i * 16, 16)] = jnp.full((16,), 1.0, jnp.float32)
        return carry
    lax.fori_loop(0, EB // 16, fill, 0)

    row0 = sid * RPS
    pltpu.sync_copy(zeros_hbm.at[0, pl.ds(row0, RPS)], acc_o.at[pl.ds(row0, RPS)])
    pltpu.sync_copy(zeros_hbm.at[1, pl.ds(row0, RPS)], acc_i.at[pl.ds(row0, RPS)])
    plsc.subcore_barrier()

    base = (cid * NS + sid) * EPW

    def body(blk, carry):
        off = pl.multiple_of(base + blk * EB, 8)
        pltpu.sync_copy(src_hbm.at[pl.ds(off, EB)], idx_s)
        pltpu.sync_copy(dst_hbm.at[pl.ds(off, EB)], idx_d)
        pltpu.sync_copy(ones_v, acc_o.at[idx_s], add=True)
        pltpu.sync_copy(ones_v, acc_i.at[idx_d], add=True)
        return carry
    lax.fori_loop(0, NBLK, body, 0)

    plsc.subcore_barrier()
    pltpu.sync_copy(acc_o.at[pl.ds(row0, RPS)], out_hbm.at[cid, 0, pl.ds(row0, RPS)])
    pltpu.sync_copy(acc_i.at[pl.ds(row0, RPS)], out_hbm.at[cid, 1, pl.ds(row0, RPS)])


@functools.partial(
    pl.kernel,
    out_type=jax.ShapeDtypeStruct((NC, NPAD, DP), jnp.float32),
    mesh=_sc_mesh,
    scratch_types=[
        pltpu.VMEM_SHARED((NPAD, DP), jnp.float32),
        pltpu.VMEM((EB,), jnp.int32),
        pltpu.VMEM((EB,), jnp.int32),
        pltpu.VMEM((EB, DP), jnp.float32),
        pltpu.SemaphoreType.DMA,
    ],
    compiler_params=pltpu.CompilerParams(use_tc_tiling_on_sc=False),
)
def _segsum_sc(feat_hbm, src_hbm, dst_hbm, zerosp_hbm, out_hbm,
               accum, idx_s, idx_d, rows, sem):
    cid = lax.axis_index("c")
    sid = lax.axis_index("s")

    row0 = sid * RPS
    pltpu.sync_copy(zerosp_hbm.at[pl.ds(row0, RPS)], accum.at[pl.ds(row0, RPS)])
    plsc.subcore_barrier()

    base = (cid * NS + sid) * EPW

    def body(blk, carry):
        off = pl.multiple_of(base + blk * EB, 8)
        pltpu.sync_copy(src_hbm.at[pl.ds(off, EB)], idx_s)
        pltpu.sync_copy(dst_hbm.at[pl.ds(off, EB)], idx_d)
        pltpu.async_copy(feat_hbm.at[idx_s], rows, sem).wait()
        pltpu.sync_copy(rows, accum.at[idx_d], add=True)
        return carry
    lax.fori_loop(0, NBLK, body, 0)

    plsc.subcore_barrier()
    pltpu.sync_copy(accum.at[pl.ds(row0, RPS)], out_hbm.at[cid, pl.ds(row0, RPS)])


def _prep_body(x_ref, wp_ref, bp_ref, degp_ref,
               h0_ref, ainv_ref, binv_ref, g1_ref):
    x = x_ref[...]
    h0 = jnp.maximum(
        jnp.dot(x, wp_ref[...], preferred_element_type=jnp.float32)
        + bp_ref[...][None, :], 0.0)
    deg = (degp_ref[0] + degp_ref[1])[:, :N]
    ainv = jnp.where(deg[0] > 0.5, lax.rsqrt(jnp.maximum(deg[0], 1.0)), 0.0)
    binv = jnp.where(deg[1] > 0.5, lax.rsqrt(jnp.maximum(deg[1], 1.0)), 0.0)
    h0_ref[...] = h0
    ainv_ref[...] = ainv
    binv_ref[...] = binv
    g1_ref[...] = jnp.concatenate(
        [h0 * ainv[:, None], jnp.zeros((N, DP - H), jnp.float32)], axis=1)


_prep_tc = pl.pallas_call(
    _prep_body,
    out_shape=(
        jax.ShapeDtypeStruct((N, H), jnp.float32),
        jax.ShapeDtypeStruct((N,), jnp.float32),
        jax.ShapeDtypeStruct((N,), jnp.float32),
        jax.ShapeDtypeStruct((N, DP), jnp.float32),
    ),
)


def _mid_body(s1p_ref, ainv_ref, binv_ref, f1_ref, g2_ref):
    s = (s1p_ref[0] + s1p_ref[1])[:N]
    ainv = ainv_ref[...]
    binv = binv_ref[...]
    f1 = s[:, :H] * binv[:, None]
    f1_ref[...] = f1
    g2_ref[...] = s * (ainv * binv)[:, None]


_mid_tc = pl.pallas_call(
    _mid_body,
    out_shape=(
        jax.ShapeDtypeStruct((N, H), jnp.float32),
        jax.ShapeDtypeStruct((N, DP), jnp.float32),
    ),
)


def _round_body(s2p_ref, ainv_ref, binv_ref, nf_ref, f1_ref, hid_ref,
                wt_ref, bt_ref, wg1_ref, bg1_ref, wg2_ref, bg2_ref,
                nf2_ref, hid2_ref, g1n_ref):
    ainv = ainv_ref[...]
    binv = binv_ref[...]
    f2 = (s2p_ref[0] + s2p_ref[1])[:N, :H] * binv[:, None]
    nf = nf_ref[...]
    f1 = f1_ref[...]
    hid = hid_ref[...]
    wt = wt_ref[...]
    t = jnp.maximum(
        jnp.dot(nf, wt[:H], preferred_element_type=jnp.float32)
        + jnp.dot(f1, wt[H:2 * H], preferred_element_type=jnp.float32)
        + jnp.dot(f2, wt[2 * H:], preferred_element_type=jnp.float32)
        + bt_ref[...][None, :], 0.0)
    wg1 = wg1_ref[...]
    wg2 = wg2_ref[...]
    nf2 = (jnp.dot(t, wg1[:H], preferred_element_type=jnp.float32)
           + jnp.dot(hid, wg1[H:], preferred_element_type=jnp.float32)
           + bg1_ref[...][None, :])
    hid2 = (jnp.dot(t, wg2[:H], preferred_element_type=jnp.float32)
            + jnp.dot(hid, wg2[H:], preferred_element_type=jnp.float32)
            + bg2_ref[...][None, :])
    nf2_ref[...] = nf2
    hid2_ref[...] = hid2
    g1n_ref[...] = jnp.concatenate(
        [nf2 * ainv[:, None], jnp.zeros((N, DP - H), jnp.float32)], axis=1)


_round_tc = pl.pallas_call(
    _round_body,
    out_shape=(
        jax.ShapeDtypeStruct((N, H), jnp.float32),
        jax.ShapeDtypeStruct((N, H), jnp.float32),
        jax.ShapeDtypeStruct((N, DP), jnp.float32),
    ),
)


def _s2s_body(h0_ref, nf_ref, wih_ref, whh_ref, bih_ref, bhh_ref,
              ws_ref, bs_ref, pa_ref, out_ref):
    feat = jnp.concatenate([h0_ref[...], nf_ref[...]], axis=1)
    wih = wih_ref[...]
    whh = whh_ref[...]
    bih = bih_ref[...][None, :]
    bhh = bhh_ref[...][None, :]
    h = jnp.zeros((1, 2 * H), jnp.float32)
    c = jnp.zeros((1, 2 * H), jnp.float32)
    q_star = jnp.zeros((1, 4 * H), jnp.float32)
    for _ in range(3):
        gates = (lax.dot_general(q_star, wih, (((1,), (1,)), ((), ())),
                                 preferred_element_type=jnp.float32) + bih
                 + lax.dot_general(h, whh, (((1,), (1,)), ((), ())),
                                   preferred_element_type=jnp.float32) + bhh)
        i_g = jax.nn.sigmoid(gates[:, 0:2 * H])
        f_g = jax.nn.sigmoid(gates[:, 2 * H:4 * H])
        g_g = jnp.tanh(gates[:, 4 * H:6 * H])
        o_g = jax.nn.sigmoid(gates[:, 6 * H:8 * H])
        c = f_g * c + i_g * g_g
        h = o_g * jnp.tanh(c)
        e = jnp.sum(feat * h, axis=-1, keepdims=True)
        m = jnp.max(e)
        p = jnp.exp(e - m)
        alpha = p / jnp.sum(p)
        readout = jnp.sum(feat * alpha, axis=0, keepdims=True)
        q_star = jnp.concatenate([h, readout], axis=-1)
    y = jnp.dot(q_star, ws_ref[...], preferred_element_type=jnp.float32) \
        + bs_ref[...][None, :]
    pa = pa_ref[0, 0]
    out_ref[...] = jnp.where(y >= 0, y, pa * y)


_s2s_tc = pl.pallas_call(
    _s2s_body,
    out_shape=jax.ShapeDtypeStruct((1, H), jnp.float32),
)


def kernel(x, edge_attr, edge_index, W_proj, b_proj, W_tag, b_tag,
           W_g1, b_g1, W_g2, b_g2, W_ih, W_hh, b_ih, b_hh, W_s, b_s, prelu_a):
    src = edge_index[0]
    dst = edge_index[1]
    zeros2 = jnp.zeros((2, NPAD), jnp.float32)
    zerosp = jnp.zeros((NPAD, DP), jnp.float32)

    degp = _degrees_sc(src, dst, zeros2)
    h0, ainv, binv, g1 = _prep_tc(x, W_proj, b_proj, degp)

    nf, hidden = h0, h0
    for _ in range(3):
        s1p = _segsum_sc(g1, src, dst, zerosp)
        f1, g2 = _mid_tc(s1p, ainv, binv)
        s2p = _segsum_sc(g2, src, dst, zerosp)
        nf, hidden, g1 = _round_tc(s2p, ainv, binv, nf, f1, hidden,
                                   W_tag, b_tag, W_g1, b_g1, W_g2, b_g2)

    return _s2s_tc(h0, nf, W_ih, W_hh, b_ih, b_hh, W_s, b_s,
                   jnp.reshape(prelu_a, (1, 1)))


# speedup vs baseline: 30.0768x; 1.6149x over previous
import functools

import jax
import jax.numpy as jnp
from jax import lax
from jax.experimental import pallas as pl
from jax.experimental.pallas import tpu as pltpu
from jax.experimental.pallas import tpu_sc as plsc

N = 10000
NPAD = 10240
E = 320000
H = 20
DP = 32
NC = 2
NS = 16
NW = NC * NS
EPW = E // NW
EB = 1000
NBLK = EPW // EB
RPS = NPAD // NS

_sc_mesh = plsc.VectorSubcoreMesh(core_axis_name="c", subcore_axis_name="s")


@functools.partial(
    pl.kernel,
    out_type=jax.ShapeDtypeStruct((NC, 2, NPAD), jnp.float32),
    mesh=_sc_mesh,
    scratch_types=[
        pltpu.VMEM_SHARED((NPAD,), jnp.float32),
        pltpu.VMEM_SHARED((NPAD,), jnp.float32),
        pltpu.VMEM((EPW,), jnp.int32),
        pltpu.VMEM((EPW,), jnp.int32),
        pltpu.VMEM((EPW,), jnp.float32),
    ],
)
def _degrees_sc(src2_hbm, dst2_hbm, ones_hbm, zeros_hbm, out_hbm,
                acc_o, acc_i, idx_s, idx_d, ones_v):
    cid = lax.axis_index("c")
    sid = lax.axis_index("s")
    wid = cid * NS + sid

    row0 = sid * RPS
    pltpu.sync_copy(zeros_hbm.at[0, pl.ds(row0, RPS)], acc_o.at[pl.ds(row0, RPS)])
    pltpu.sync_copy(zeros_hbm.at[1, pl.ds(row0, RPS)], acc_i.at[pl.ds(row0, RPS)])
    pltpu.sync_copy(src2_hbm.at[wid], idx_s)
    pltpu.sync_copy(dst2_hbm.at[wid], idx_d)
    pltpu.sync_copy(ones_hbm, ones_v)
    plsc.subcore_barrier()

    pltpu.sync_copy(ones_v, acc_o.at[idx_s], add=True)
    pltpu.sync_copy(ones_v, acc_i.at[idx_d], add=True)

    plsc.subcore_barrier()
    pltpu.sync_copy(acc_o.at[pl.ds(row0, RPS)], out_hbm.at[cid, 0, pl.ds(row0, RPS)])
    pltpu.sync_copy(acc_i.at[pl.ds(row0, RPS)], out_hbm.at[cid, 1, pl.ds(row0, RPS)])


@functools.partial(
    pl.kernel,
    out_type=jax.ShapeDtypeStruct((NC, NPAD, DP), jnp.float32),
    mesh=_sc_mesh,
    scratch_types=[
        pltpu.VMEM_SHARED((NPAD, DP), jnp.float32),
        pltpu.VMEM((NBLK, EB), jnp.int32),
        pltpu.VMEM((NBLK, EB), jnp.int32),
        pltpu.VMEM((EB, DP), jnp.float32),
        pltpu.VMEM((EB, DP), jnp.float32),
        pltpu.SemaphoreType.DMA,
        pltpu.SemaphoreType.DMA,
    ],
    compiler_params=pltpu.CompilerParams(use_tc_tiling_on_sc=False),
)
def _segsum_sc(feat_hbm, src3_hbm, dst3_hbm, zerosp_hbm, out_hbm,
               accum, srcv, dstv, rows0, rows1, sem0, sem1):
    cid = lax.axis_index("c")
    sid = lax.axis_index("s")
    wid = cid * NS + sid

    row0 = sid * RPS
    pltpu.sync_copy(zerosp_hbm.at[pl.ds(row0, RPS)], accum.at[pl.ds(row0, RPS)])
    pltpu.sync_copy(src3_hbm.at[wid], srcv)
    pltpu.sync_copy(dst3_hbm.at[wid], dstv)
    plsc.subcore_barrier()

    pltpu.async_copy(feat_hbm.at[srcv.at[0]], rows0, sem0)

    def body(i, carry):
        b0 = 2 * i
        pltpu.async_copy(feat_hbm.at[srcv.at[b0 + 1]], rows1, sem1)
        pltpu.make_async_copy(feat_hbm.at[srcv.at[b0]], rows0, sem0).wait()
        pltpu.sync_copy(rows0, accum.at[dstv.at[b0]], add=True)

        @pl.when(i < NBLK // 2 - 1)
        def _():
            pltpu.async_copy(feat_hbm.at[srcv.at[b0 + 2]], rows0, sem0)

        pltpu.make_async_copy(feat_hbm.at[srcv.at[b0 + 1]], rows1, sem1).wait()
        pltpu.sync_copy(rows1, accum.at[dstv.at[b0 + 1]], add=True)
        return carry
    lax.fori_loop(0, NBLK // 2, body, 0)

    plsc.subcore_barrier()
    pltpu.sync_copy(accum.at[pl.ds(row0, RPS)], out_hbm.at[cid, pl.ds(row0, RPS)])


def _prep_body(x_ref, wp_ref, bp_ref, degp_ref,
               h0_ref, ainv_ref, binv_ref, g1_ref):
    x = x_ref[...]
    h0 = jnp.maximum(
        jnp.dot(x, wp_ref[...], preferred_element_type=jnp.float32)
        + bp_ref[...][None, :], 0.0)
    deg = (degp_ref[0] + degp_ref[1])[:, :N]
    ainv = jnp.where(deg[0] > 0.5, lax.rsqrt(jnp.maximum(deg[0], 1.0)), 0.0)
    binv = jnp.where(deg[1] > 0.5, lax.rsqrt(jnp.maximum(deg[1], 1.0)), 0.0)
    h0_ref[...] = h0
    ainv_ref[...] = ainv
    binv_ref[...] = binv
    g1_ref[...] = jnp.concatenate(
        [h0 * ainv[:, None], jnp.zeros((N, DP - H), jnp.float32)], axis=1)


_prep_tc = pl.pallas_call(
    _prep_body,
    out_shape=(
        jax.ShapeDtypeStruct((N, H), jnp.float32),
        jax.ShapeDtypeStruct((N,), jnp.float32),
        jax.ShapeDtypeStruct((N,), jnp.float32),
        jax.ShapeDtypeStruct((N, DP), jnp.float32),
    ),
)


def _mid_body(s1p_ref, ainv_ref, binv_ref, f1_ref, g2_ref):
    s = (s1p_ref[0] + s1p_ref[1])[:N]
    ainv = ainv_ref[...]
    binv = binv_ref[...]
    f1 = s[:, :H] * binv[:, None]
    f1_ref[...] = f1
    g2_ref[...] = s * (ainv * binv)[:, None]


_mid_tc = pl.pallas_call(
    _mid_body,
    out_shape=(
        jax.ShapeDtypeStruct((N, H), jnp.float32),
        jax.ShapeDtypeStruct((N, DP), jnp.float32),
    ),
)


def _round_body(s2p_ref, ainv_ref, binv_ref, nf_ref, f1_ref, hid_ref,
                wt_ref, bt_ref, wg1_ref, bg1_ref, wg2_ref, bg2_ref,
                nf2_ref, hid2_ref, g1n_ref):
    ainv = ainv_ref[...]
    binv = binv_ref[...]
    f2 = (s2p_ref[0] + s2p_ref[1])[:N, :H] * binv[:, None]
    nf = nf_ref[...]
    f1 = f1_ref[...]
    hid = hid_ref[...]
    wt = wt_ref[...]
    t = jnp.maximum(
        jnp.dot(nf, wt[:H], preferred_element_type=jnp.float32)
        + jnp.dot(f1, wt[H:2 * H], preferred_element_type=jnp.float32)
        + jnp.dot(f2, wt[2 * H:], preferred_element_type=jnp.float32)
        + bt_ref[...][None, :], 0.0)
    wg1 = wg1_ref[...]
    wg2 = wg2_ref[...]
    nf2 = (jnp.dot(t, wg1[:H], preferred_element_type=jnp.float32)
           + jnp.dot(hid, wg1[H:], preferred_element_type=jnp.float32)
           + bg1_ref[...][None, :])
    hid2 = (jnp.dot(t, wg2[:H], preferred_element_type=jnp.float32)
            + jnp.dot(hid, wg2[H:], preferred_element_type=jnp.float32)
            + bg2_ref[...][None, :])
    nf2_ref[...] = nf2
    hid2_ref[...] = hid2
    g1n_ref[...] = jnp.concatenate(
        [nf2 * ainv[:, None], jnp.zeros((N, DP - H), jnp.float32)], axis=1)


_round_tc = pl.pallas_call(
    _round_body,
    out_shape=(
        jax.ShapeDtypeStruct((N, H), jnp.float32),
        jax.ShapeDtypeStruct((N, H), jnp.float32),
        jax.ShapeDtypeStruct((N, DP), jnp.float32),
    ),
)


def _s2s_body(h0_ref, nf_ref, wih_ref, whh_ref, bih_ref, bhh_ref,
              ws_ref, bs_ref, pa_ref, out_ref):
    feat = jnp.concatenate([h0_ref[...], nf_ref[...]], axis=1)
    wih = wih_ref[...]
    whh = whh_ref[...]
    bih = bih_ref[...][None, :]
    bhh = bhh_ref[...][None, :]
    h = jnp.zeros((1, 2 * H), jnp.float32)
    c = jnp.zeros((1, 2 * H), jnp.float32)
    q_star = jnp.zeros((1, 4 * H), jnp.float32)
    for _ in range(3):
        gates = (lax.dot_general(q_star, wih, (((1,), (1,)), ((), ())),
                                 preferred_element_type=jnp.float32) + bih
                 + lax.dot_general(h, whh, (((1,), (1,)), ((), ())),
                                   preferred_element_type=jnp.float32) + bhh)
        i_g = jax.nn.sigmoid(gates[:, 0:2 * H])
        f_g = jax.nn.sigmoid(gates[:, 2 * H:4 * H])
        g_g = jnp.tanh(gates[:, 4 * H:6 * H])
        o_g = jax.nn.sigmoid(gates[:, 6 * H:8 * H])
        c = f_g * c + i_g * g_g
        h = o_g * jnp.tanh(c)
        e = jnp.sum(feat * h, axis=-1, keepdims=True)
        m = jnp.max(e)
        p = jnp.exp(e - m)
        alpha = p / jnp.sum(p)
        readout = jnp.sum(feat * alpha, axis=0, keepdims=True)
        q_star = jnp.concatenate([h, readout], axis=-1)
    y = jnp.dot(q_star, ws_ref[...], preferred_element_type=jnp.float32) \
        + bs_ref[...][None, :]
    pa = pa_ref[0, 0]
    out_ref[...] = jnp.where(y >= 0, y, pa * y)


_s2s_tc = pl.pallas_call(
    _s2s_body,
    out_shape=jax.ShapeDtypeStruct((1, H), jnp.float32),
)


def kernel(x, edge_attr, edge_index, W_proj, b_proj, W_tag, b_tag,
           W_g1, b_g1, W_g2, b_g2, W_ih, W_hh, b_ih, b_hh, W_s, b_s, prelu_a):
    src = edge_index[0]
    dst = edge_index[1]
    src2 = jnp.reshape(src, (NW, EPW))
    dst2 = jnp.reshape(dst, (NW, EPW))
    src3 = jnp.reshape(src, (NW, NBLK, EB))
    dst3 = jnp.reshape(dst, (NW, NBLK, EB))
    ones1 = jnp.ones((EPW,), jnp.float32)
    zeros2 = jnp.zeros((2, NPAD), jnp.float32)
    zerosp = jnp.zeros((NPAD, DP), jnp.float32)

    degp = _degrees_sc(src2, dst2, ones1, zeros2)
    h0, ainv, binv, g1 = _prep_tc(x, W_proj, b_proj, degp)

    nf, hidden = h0, h0
    for _ in range(3):
        s1p = _segsum_sc(g1, src3, dst3, zerosp)
        f1, g2 = _mid_tc(s1p, ainv, binv)
        s2p = _segsum_sc(g2, src3, dst3, zerosp)
        nf, hidden, g1 = _round_tc(s2p, ainv, binv, nf, f1, hidden,
                                   W_tag, b_tag, W_g1, b_g1, W_g2, b_g2)

    return _s2s_tc(h0, nf, W_ih, W_hh, b_ih, b_hh, W_s, b_s,
                   jnp.reshape(prelu_a, (1, 1)))
